# baseline (speedup 1.0000x reference)
import functools

import jax
import jax.numpy as jnp
from jax import lax
from jax.experimental import pallas as pl
from jax.experimental.pallas import tpu as pltpu
from jax.experimental.pallas import tpu_sc as plsc

NUM_WORKERS = 32
IDX_BLK = 1024
SUB = 128
NSUB = IDX_BLK // SUB


@functools.lru_cache(maxsize=None)
def _build_gather(n: int, v: int, d: int):
    b_per_w = n // NUM_WORKERS
    n_blk = b_per_w // IDX_BLK
    assert b_per_w * NUM_WORKERS == n and n_blk * IDX_BLK == b_per_w

    mesh = plsc.VectorSubcoreMesh(core_axis_name="c", subcore_axis_name="s")

    @functools.partial(
        pl.kernel,
        mesh=mesh,
        out_type=jax.ShapeDtypeStruct((n, d), jnp.float32),
        scratch_types=[
            pltpu.VMEM((IDX_BLK,), jnp.int32),
            pltpu.VMEM((IDX_BLK, d), jnp.float32),
            pltpu.SemaphoreType.DMA,
        ],
        compiler_params=pltpu.CompilerParams(use_tc_tiling_on_sc=False),
    )
    def gather_kernel(table_hbm, idx_hbm, out_hbm, idx_v, rows_v, sem):
        wid = lax.axis_index("s") * 2 + lax.axis_index("c")
        base = wid * b_per_w

        def blk_body(b, carry):
            off = base + b * IDX_BLK
            pltpu.sync_copy(idx_hbm.at[pl.ds(off, IDX_BLK)], idx_v)
            copies = []
            for j in range(NSUB):
                copies.append(pltpu.async_copy(
                    table_hbm.at[idx_v.at[pl.ds(j * SUB, SUB)]],
                    rows_v.at[pl.ds(j * SUB, SUB)],
                    sem,
                ))
            for c in copies:
                c.wait()
            pltpu.sync_copy(rows_v, out_hbm.at[pl.ds(off, IDX_BLK)])
            return carry

        lax.fori_loop(0, n_blk, blk_body, 0)

    return gather_kernel


def kernel(input_index, embedding_weight):
    b, s = input_index.shape
    v, d = embedding_weight.shape
    n = b * s
    idx_flat = input_index.reshape(n).astype(jnp.int32)
    out = _build_gather(n, v, d)(embedding_weight, idx_flat)
    return out.reshape(b, s, d)


# speedup vs baseline: 1.0126x; 1.0126x over previous
import functools

import jax
import jax.numpy as jnp
from jax import lax
from jax.experimental import pallas as pl
from jax.experimental.pallas import tpu as pltpu
from jax.experimental.pallas import tpu_sc as plsc

NUM_WORKERS = 32
BLK = 512
SUB = 128
NSUB = BLK // SUB


@functools.lru_cache(maxsize=None)
def _build_gather(n: int, v: int, d: int):
    b_per_w = n // NUM_WORKERS
    n_blk = b_per_w // BLK
    assert b_per_w * NUM_WORKERS == n and n_blk * BLK == b_per_w
    assert n_blk % 2 == 0

    mesh = plsc.VectorSubcoreMesh(core_axis_name="c", subcore_axis_name="s")

    @functools.partial(
        pl.kernel,
        mesh=mesh,
        out_type=jax.ShapeDtypeStruct((n, d), jnp.float32),
        scratch_types=[
            pltpu.VMEM((b_per_w,), jnp.int32),
            pltpu.VMEM((2 * BLK, d), jnp.float32),
            pltpu.SemaphoreType.DMA,
            pltpu.SemaphoreType.DMA,
            pltpu.SemaphoreType.DMA,
            pltpu.SemaphoreType.DMA,
        ],
        compiler_params=pltpu.CompilerParams(use_tc_tiling_on_sc=False),
    )
    def gather_kernel(table_hbm, idx_hbm, out_hbm, idx_v, rows_v, sem_ga,
                      sem_gb, sem_wa, sem_wb):
        wid = lax.axis_index("s") * 2 + lax.axis_index("c")
        base = wid * b_per_w
        g_sems = (sem_ga, sem_gb)
        w_sems = (sem_wa, sem_wb)

        def fire_gathers(blk, half):
            for j in range(NSUB):
                pltpu.async_copy(
                    table_hbm.at[idx_v.at[pl.ds(blk * BLK + j * SUB, SUB)]],
                    rows_v.at[pl.ds(half * BLK + j * SUB, SUB)],
                    g_sems[half],
                )

        def drain_gathers(half):
            pltpu.make_async_copy(
                table_hbm.at[pl.ds(0, BLK)],
                rows_v.at[pl.ds(half * BLK, BLK)],
                g_sems[half],
            ).wait()

        def fire_write(blk, half):
            pltpu.async_copy(
                rows_v.at[pl.ds(half * BLK, BLK)],
                out_hbm.at[pl.ds(base + blk * BLK, BLK)],
                w_sems[half],
            )

        def drain_write(half):
            pltpu.make_async_copy(
                rows_v.at[pl.ds(half * BLK, BLK)],
                out_hbm.at[pl.ds(base, BLK)],
                w_sems[half],
            ).wait()

        pltpu.sync_copy(idx_hbm.at[pl.ds(base, b_per_w)], idx_v)
        fire_gathers(0, 0)
        fire_gathers(1, 1)

        def body(g, carry):
            b = 2 * g
            drain_gathers(0)
            fire_write(b, 0)
            drain_gathers(1)
            fire_write(b + 1, 1)
            drain_write(0)
            fire_gathers(b + 2, 0)
            drain_write(1)
            fire_gathers(b + 3, 1)
            return carry

        lax.fori_loop(0, n_blk // 2 - 1, body, 0)

        drain_gathers(0)
        fire_write(n_blk - 2, 0)
        drain_gathers(1)
        fire_write(n_blk - 1, 1)
        drain_write(0)
        drain_write(1)

    return gather_kernel


def kernel(input_index, embedding_weight):
    b, s = input_index.shape
    v, d = embedding_weight.shape
    n = b * s
    idx_flat = input_index.reshape(n).astype(jnp.int32)
    out = _build_gather(n, v, d)(embedding_weight, idx_flat)
    return out.reshape(b, s, d)
